# baseline (speedup 1.0000x reference)
import functools

import jax
import jax.numpy as jnp
from jax.experimental import pallas as pl
from jax.experimental.pallas import tpu as pltpu

N_ITERS = 3
B = 64
BLK = 16384
NEG = -1e30


def _body(nb, n, x_ref, bt_ref, wi_ref, wh_ref, b_ref, out_ref,
          h_ref, c_ref, q_ref, qst_ref, m_ref, s_ref, v_ref):
    i = pl.program_id(0)
    t = i // nb
    j = i % nb
    D = q_ref.shape[1]

    @pl.when(i == 0)
    def _init():
        h_ref[...] = jnp.zeros_like(h_ref)
        c_ref[...] = jnp.zeros_like(c_ref)
        qst_ref[...] = jnp.zeros_like(qst_ref)

    @pl.when(j == 0)
    def _iter_start():
        @pl.when(t > 0)
        def _():
            recip = 1.0 / (s_ref[...] + 1e-16)
            vn = v_ref[...] * recip
            eye = (jax.lax.broadcasted_iota(jnp.int32, (B, B), 0)
                   == jax.lax.broadcasted_iota(jnp.int32, (B, B), 1)
                   ).astype(jnp.float32)
            readout = jax.lax.dot_general(
                eye, vn, (((1,), (1,)), ((), ())),
                preferred_element_type=jnp.float32)
            qst_ref[:, :D] = q_ref[...]
            qst_ref[:, D:] = readout

        @pl.when(t == N_ITERS)
        def _():
            out_ref[...] = qst_ref[...]

        @pl.when(t < N_ITERS)
        def _():
            gates = (
                jax.lax.dot_general(qst_ref[...], wi_ref[...],
                                    (((1,), (0,)), ((), ())),
                                    preferred_element_type=jnp.float32)
                + jax.lax.dot_general(h_ref[...], wh_ref[...],
                                      (((1,), (0,)), ((), ())),
                                      preferred_element_type=jnp.float32)
                + b_ref[...])
            ig = jax.nn.sigmoid(gates[:, 0 * D:1 * D])
            fg = jax.nn.sigmoid(gates[:, 1 * D:2 * D])
            gg = jnp.tanh(gates[:, 2 * D:3 * D])
            og = jax.nn.sigmoid(gates[:, 3 * D:4 * D])
            c_new = fg * c_ref[...] + ig * gg
            c_ref[...] = c_new
            h_new = og * jnp.tanh(c_new)
            h_ref[...] = h_new
            q_ref[...] = h_new
            m_ref[...] = jnp.full_like(m_ref, NEG)
            s_ref[...] = jnp.zeros_like(s_ref)
            v_ref[...] = jnp.zeros_like(v_ref)

    def _process(xv, onehot):
        xq = jax.lax.dot_general(xv, q_ref[...],
                                 (((1,), (1,)), ((), ())),
                                 preferred_element_type=jnp.float32)
        m_blk = jnp.max(jnp.where(onehot, xq, NEG), axis=0, keepdims=True)
        m_old = m_ref[...]
        m_new = jnp.maximum(m_old, m_blk)
        w = jnp.where(onehot, jnp.exp(xq - m_new), 0.0)
        scale = jnp.exp(m_old - m_new)
        s_ref[...] = s_ref[...] * scale + jnp.sum(w, axis=0, keepdims=True)
        v_ref[...] = v_ref[...] * scale + jax.lax.dot_general(
            xv, w, (((0,), (0,)), ((), ())),
            preferred_element_type=jnp.float32)
        m_ref[...] = m_new

    base = j * BLK
    is_tail = base + BLK > n
    iota_b = jax.lax.broadcasted_iota(jnp.int32, (BLK, B), 1)

    @pl.when(jnp.logical_and(t < N_ITERS, jnp.logical_not(is_tail)))
    def _block_full():
        _process(x_ref[...], bt_ref[...] == iota_b)

    @pl.when(jnp.logical_and(t < N_ITERS, is_tail))
    def _block_tail():
        valid = (jax.lax.broadcasted_iota(jnp.int32, (BLK, 1), 0) + base) < n
        xv = jnp.where(valid, x_ref[...], 0.0)
        _process(xv, jnp.logical_and(bt_ref[...] == iota_b, valid))


@functools.partial(jax.jit, static_argnames=())
def kernel(x, batch, W_ih, W_hh, b_ih, b_hh):
    n, d = x.shape
    nb = (n + BLK - 1) // BLK
    bt = batch.astype(jnp.int32).reshape(n, 1)
    wi_t = W_ih.T
    wh_t = W_hh.T
    bias = (b_ih + b_hh).reshape(1, 4 * d)

    grid = (N_ITERS * nb + 1,)
    out = pl.pallas_call(
        functools.partial(_body, nb, n),
        grid=grid,
        in_specs=[
            pl.BlockSpec((BLK, d), lambda i: (i % nb, 0)),
            pl.BlockSpec((BLK, 1), lambda i: (i % nb, 0)),
            pl.BlockSpec(wi_t.shape, lambda i: (0, 0)),
            pl.BlockSpec(wh_t.shape, lambda i: (0, 0)),
            pl.BlockSpec(bias.shape, lambda i: (0, 0)),
        ],
        out_specs=pl.BlockSpec((B, 2 * d), lambda i: (0, 0)),
        out_shape=jax.ShapeDtypeStruct((B, 2 * d), jnp.float32),
        scratch_shapes=[
            pltpu.VMEM((B, d), jnp.float32),
            pltpu.VMEM((B, d), jnp.float32),
            pltpu.VMEM((B, d), jnp.float32),
            pltpu.VMEM((B, 2 * d), jnp.float32),
            pltpu.VMEM((1, B), jnp.float32),
            pltpu.VMEM((1, B), jnp.float32),
            pltpu.VMEM((d, B), jnp.float32),
        ],
        compiler_params=pltpu.CompilerParams(
            dimension_semantics=("arbitrary",)),
    )(x, bt, wi_t, wh_t, bias)
    return out


# speedup vs baseline: 1.0346x; 1.0346x over previous
import functools

import jax
import jax.numpy as jnp
from jax.experimental import pallas as pl
from jax.experimental.pallas import tpu as pltpu

N_ITERS = 3
B = 64
BLK = 4096
NEG = -1e30


def _body(nb, n, x_ref, bt_ref, wi_ref, wh_ref, b_ref, out_ref,
          h_ref, c_ref, q_ref, qst_ref, m_ref, s_ref, v_ref):
    i = pl.program_id(0)
    t = i // nb
    j = i % nb
    D = q_ref.shape[1]

    @pl.when(i == 0)
    def _init():
        h_ref[...] = jnp.zeros_like(h_ref)
        c_ref[...] = jnp.zeros_like(c_ref)
        qst_ref[...] = jnp.zeros_like(qst_ref)

    @pl.when(j == 0)
    def _iter_start():
        @pl.when(t > 0)
        def _():
            recip = 1.0 / (s_ref[...] + 1e-16)
            vn = v_ref[...] * recip
            eye = (jax.lax.broadcasted_iota(jnp.int32, (B, B), 0)
                   == jax.lax.broadcasted_iota(jnp.int32, (B, B), 1)
                   ).astype(jnp.float32)
            readout = jax.lax.dot_general(
                eye, vn, (((1,), (1,)), ((), ())),
                preferred_element_type=jnp.float32)
            qst_ref[:, :D] = q_ref[...]
            qst_ref[:, D:] = readout

        @pl.when(t == N_ITERS)
        def _():
            out_ref[...] = qst_ref[...]

        @pl.when(t < N_ITERS)
        def _():
            gates = (
                jax.lax.dot_general(qst_ref[...], wi_ref[...],
                                    (((1,), (0,)), ((), ())),
                                    preferred_element_type=jnp.float32)
                + jax.lax.dot_general(h_ref[...], wh_ref[...],
                                      (((1,), (0,)), ((), ())),
                                      preferred_element_type=jnp.float32)
                + b_ref[...])
            ig = jax.nn.sigmoid(gates[:, 0 * D:1 * D])
            fg = jax.nn.sigmoid(gates[:, 1 * D:2 * D])
            gg = jnp.tanh(gates[:, 2 * D:3 * D])
            og = jax.nn.sigmoid(gates[:, 3 * D:4 * D])
            c_new = fg * c_ref[...] + ig * gg
            c_ref[...] = c_new
            h_new = og * jnp.tanh(c_new)
            h_ref[...] = h_new
            q_ref[...] = h_new
            m_ref[...] = jnp.full_like(m_ref, NEG)
            s_ref[...] = jnp.zeros_like(s_ref)
            v_ref[...] = jnp.zeros_like(v_ref)

    def _process(xv, onehot):
        xq = jax.lax.dot_general(xv, q_ref[...],
                                 (((1,), (1,)), ((), ())),
                                 preferred_element_type=jnp.float32)
        m_blk = jnp.max(jnp.where(onehot, xq, NEG), axis=0, keepdims=True)
        m_old = m_ref[...]
        m_new = jnp.maximum(m_old, m_blk)
        w = jnp.where(onehot, jnp.exp(xq - m_new), 0.0)
        scale = jnp.exp(m_old - m_new)
        s_ref[...] = s_ref[...] * scale + jnp.sum(w, axis=0, keepdims=True)
        v_ref[...] = v_ref[...] * scale + jax.lax.dot_general(
            xv, w, (((0,), (0,)), ((), ())),
            preferred_element_type=jnp.float32)
        m_ref[...] = m_new

    base = j * BLK
    is_tail = base + BLK > n
    iota_b = jax.lax.broadcasted_iota(jnp.int32, (BLK, B), 1)

    @pl.when(jnp.logical_and(t < N_ITERS, jnp.logical_not(is_tail)))
    def _block_full():
        _process(x_ref[...], bt_ref[...] == iota_b)

    @pl.when(jnp.logical_and(t < N_ITERS, is_tail))
    def _block_tail():
        valid = (jax.lax.broadcasted_iota(jnp.int32, (BLK, 1), 0) + base) < n
        xv = jnp.where(valid, x_ref[...], 0.0)
        _process(xv, jnp.logical_and(bt_ref[...] == iota_b, valid))


@functools.partial(jax.jit, static_argnames=())
def kernel(x, batch, W_ih, W_hh, b_ih, b_hh):
    n, d = x.shape
    nb = (n + BLK - 1) // BLK
    bt = batch.astype(jnp.int32).reshape(n, 1)
    wi_t = W_ih.T
    wh_t = W_hh.T
    bias = (b_ih + b_hh).reshape(1, 4 * d)

    grid = (N_ITERS * nb + 1,)
    out = pl.pallas_call(
        functools.partial(_body, nb, n),
        grid=grid,
        in_specs=[
            pl.BlockSpec((BLK, d), lambda i: (i % nb, 0)),
            pl.BlockSpec((BLK, 1), lambda i: (i % nb, 0)),
            pl.BlockSpec(wi_t.shape, lambda i: (0, 0)),
            pl.BlockSpec(wh_t.shape, lambda i: (0, 0)),
            pl.BlockSpec(bias.shape, lambda i: (0, 0)),
        ],
        out_specs=pl.BlockSpec((B, 2 * d), lambda i: (0, 0)),
        out_shape=jax.ShapeDtypeStruct((B, 2 * d), jnp.float32),
        scratch_shapes=[
            pltpu.VMEM((B, d), jnp.float32),
            pltpu.VMEM((B, d), jnp.float32),
            pltpu.VMEM((B, d), jnp.float32),
            pltpu.VMEM((B, 2 * d), jnp.float32),
            pltpu.VMEM((1, B), jnp.float32),
            pltpu.VMEM((1, B), jnp.float32),
            pltpu.VMEM((d, B), jnp.float32),
        ],
        compiler_params=pltpu.CompilerParams(
            dimension_semantics=("arbitrary",)),
    )(x, bt, wi_t, wh_t, bias)
    return out


# speedup vs baseline: 1.0350x; 1.0003x over previous
import functools

import jax
import jax.numpy as jnp
from jax.experimental import pallas as pl
from jax.experimental.pallas import tpu as pltpu

N_ITERS = 3
B = 64
BLK = 8192
NEG = -1e30


def _body(nb, n, x_ref, bt_ref, wi_ref, wh_ref, b_ref, out_ref,
          h_ref, c_ref, q_ref, qst_ref, m_ref, s_ref, v_ref):
    i = pl.program_id(0)
    t = i // nb
    j = i % nb
    D = q_ref.shape[1]

    @pl.when(i == 0)
    def _init():
        h_ref[...] = jnp.zeros_like(h_ref)
        c_ref[...] = jnp.zeros_like(c_ref)
        qst_ref[...] = jnp.zeros_like(qst_ref)

    @pl.when(j == 0)
    def _iter_start():
        @pl.when(t > 0)
        def _():
            recip = 1.0 / (s_ref[...] + 1e-16)
            vn = v_ref[...] * recip
            eye = (jax.lax.broadcasted_iota(jnp.int32, (B, B), 0)
                   == jax.lax.broadcasted_iota(jnp.int32, (B, B), 1)
                   ).astype(jnp.float32)
            readout = jax.lax.dot_general(
                eye, vn, (((1,), (1,)), ((), ())),
                preferred_element_type=jnp.float32)
            qst_ref[:, :D] = q_ref[...]
            qst_ref[:, D:] = readout

        @pl.when(t == N_ITERS)
        def _():
            out_ref[...] = qst_ref[...]

        @pl.when(t < N_ITERS)
        def _():
            gates = (
                jax.lax.dot_general(qst_ref[...], wi_ref[...],
                                    (((1,), (0,)), ((), ())),
                                    preferred_element_type=jnp.float32)
                + jax.lax.dot_general(h_ref[...], wh_ref[...],
                                      (((1,), (0,)), ((), ())),
                                      preferred_element_type=jnp.float32)
                + b_ref[...])
            ig = jax.nn.sigmoid(gates[:, 0 * D:1 * D])
            fg = jax.nn.sigmoid(gates[:, 1 * D:2 * D])
            gg = jnp.tanh(gates[:, 2 * D:3 * D])
            og = jax.nn.sigmoid(gates[:, 3 * D:4 * D])
            c_new = fg * c_ref[...] + ig * gg
            c_ref[...] = c_new
            h_new = og * jnp.tanh(c_new)
            h_ref[...] = h_new
            q_ref[...] = h_new
            m_ref[...] = jnp.full_like(m_ref, NEG)
            s_ref[...] = jnp.zeros_like(s_ref)
            v_ref[...] = jnp.zeros_like(v_ref)

    def _process(xv, onehot):
        xq = jax.lax.dot_general(xv, q_ref[...],
                                 (((1,), (1,)), ((), ())),
                                 preferred_element_type=jnp.float32)
        m_blk = jnp.max(jnp.where(onehot, xq, NEG), axis=0, keepdims=True)
        m_old = m_ref[...]
        m_new = jnp.maximum(m_old, m_blk)
        w = jnp.where(onehot, jnp.exp(xq - m_new), 0.0)
        scale = jnp.exp(m_old - m_new)
        s_ref[...] = s_ref[...] * scale + jnp.sum(w, axis=0, keepdims=True)
        v_ref[...] = v_ref[...] * scale + jax.lax.dot_general(
            xv, w, (((0,), (0,)), ((), ())),
            preferred_element_type=jnp.float32)
        m_ref[...] = m_new

    base = j * BLK
    is_tail = base + BLK > n
    iota_b = jax.lax.broadcasted_iota(jnp.int32, (BLK, B), 1)

    @pl.when(jnp.logical_and(t < N_ITERS, jnp.logical_not(is_tail)))
    def _block_full():
        _process(x_ref[...], bt_ref[...] == iota_b)

    @pl.when(jnp.logical_and(t < N_ITERS, is_tail))
    def _block_tail():
        valid = (jax.lax.broadcasted_iota(jnp.int32, (BLK, 1), 0) + base) < n
        xv = jnp.where(valid, x_ref[...], 0.0)
        _process(xv, jnp.logical_and(bt_ref[...] == iota_b, valid))


@functools.partial(jax.jit, static_argnames=())
def kernel(x, batch, W_ih, W_hh, b_ih, b_hh):
    n, d = x.shape
    nb = (n + BLK - 1) // BLK
    bt = batch.astype(jnp.int32).reshape(n, 1)
    wi_t = W_ih.T
    wh_t = W_hh.T
    bias = (b_ih + b_hh).reshape(1, 4 * d)

    grid = (N_ITERS * nb + 1,)
    out = pl.pallas_call(
        functools.partial(_body, nb, n),
        grid=grid,
        in_specs=[
            pl.BlockSpec((BLK, d), lambda i: (i % nb, 0)),
            pl.BlockSpec((BLK, 1), lambda i: (i % nb, 0)),
            pl.BlockSpec(wi_t.shape, lambda i: (0, 0)),
            pl.BlockSpec(wh_t.shape, lambda i: (0, 0)),
            pl.BlockSpec(bias.shape, lambda i: (0, 0)),
        ],
        out_specs=pl.BlockSpec((B, 2 * d), lambda i: (0, 0)),
        out_shape=jax.ShapeDtypeStruct((B, 2 * d), jnp.float32),
        scratch_shapes=[
            pltpu.VMEM((B, d), jnp.float32),
            pltpu.VMEM((B, d), jnp.float32),
            pltpu.VMEM((B, d), jnp.float32),
            pltpu.VMEM((B, 2 * d), jnp.float32),
            pltpu.VMEM((1, B), jnp.float32),
            pltpu.VMEM((1, B), jnp.float32),
            pltpu.VMEM((d, B), jnp.float32),
        ],
        compiler_params=pltpu.CompilerParams(
            dimension_semantics=("arbitrary",)),
    )(x, bt, wi_t, wh_t, bias)
    return out
